# baseline (speedup 1.0000x reference)
import functools

import jax
import jax.numpy as jnp
from jax import lax
from jax.experimental import pallas as pl
from jax.experimental.pallas import tpu as pltpu
from jax.experimental.pallas import tpu_sc as plsc

_N = 10000
_E = 320000
_H = 128
_G = 128
_ED = 16
_OUT = 32

_NC = 2
_NS = 16
_NW = _NC * _NS
_K = 128
_CH = 80
_EPAD = _NW * _CH * _K
_NACC = 10112
_ZR = _NACC // _NS
_BR = 2000


def _sc_mesh():
    return plsc.VectorSubcoreMesh(
        core_axis_name="c", subcore_axis_name="s",
        num_cores=_NC, num_subcores=_NS)


def _deg_body(col_hbm, out_hbm, col_v, hist):
    cid = lax.axis_index("c")
    sid = lax.axis_index("s")
    wid = cid * _NS + sid
    pltpu.sync_copy(col_hbm.at[wid], col_v)
    zero16 = jnp.zeros((16,), jnp.float32)
    one16 = jnp.ones((16,), jnp.float32)

    def zbody(i, c):
        hist[pl.ds(i * 16, 16)] = zero16
        return c

    lax.fori_loop(0, _NACC // 16, zbody, 0)

    def body(j, c):
        def inner(k, c2):
            idx = col_v[j, pl.ds(k * 16, 16)]
            plsc.addupdate_scatter(hist, [idx], one16)
            return c2

        lax.fori_loop(0, _K // 16, inner, 0)
        return c

    lax.fori_loop(0, _CH, body, 0)
    pltpu.sync_copy(hist, out_hbm.at[wid])


@functools.cache
def _deg_pass():
    return pl.kernel(
        _deg_body,
        out_type=jax.ShapeDtypeStruct((_NW, _NACC), jnp.float32),
        mesh=_sc_mesh(),
        compiler_params=pltpu.CompilerParams(needs_layout_passes=False),
        scratch_types=[
            pltpu.VMEM((_CH, _K), jnp.int32),
            pltpu.VMEM((_NACC,), jnp.float32),
        ],
    )


def _edge_body(z_hbm, row_hbm, col_hbm, zero_hbm, out_hbm,
               row_v, col_v, buf, acc, sem):
    cid = lax.axis_index("c")
    sid = lax.axis_index("s")
    wid = cid * _NS + sid
    pltpu.sync_copy(zero_hbm, acc.at[pl.ds(sid * _ZR, _ZR)])
    pltpu.sync_copy(row_hbm.at[wid], row_v)
    pltpu.sync_copy(col_hbm.at[wid], col_v)
    plsc.subcore_barrier()

    def body(j, carry):
        pltpu.async_copy(z_hbm.at[row_v.at[j]], buf, sem).wait()
        pltpu.sync_copy(buf, acc.at[col_v.at[j]], add=True)
        return carry

    lax.fori_loop(0, _CH, body, 0)
    plsc.subcore_barrier()
    pltpu.sync_copy(acc.at[pl.ds(sid * _ZR, _ZR)],
                    out_hbm.at[cid].at[pl.ds(sid * _ZR, _ZR)])


@functools.cache
def _edge_pass():
    return pl.kernel(
        _edge_body,
        out_type=jax.ShapeDtypeStruct((_NC, _NACC, _H), jnp.float32),
        mesh=_sc_mesh(),
        scratch_types=[
            pltpu.VMEM((_CH, _K), jnp.int32),
            pltpu.VMEM((_CH, _K), jnp.int32),
            pltpu.VMEM((_K, _H), jnp.float32),
            pltpu.VMEM_SHARED((_NACC, _H), jnp.float32),
            pltpu.SemaphoreType.DMA,
        ],
    )


def _prep_body(x_ref, w_ref, d_ref, z_ref, dis_ref):
    deg = jnp.sum(d_ref[...], axis=1, keepdims=True) + 1.0
    dis = lax.rsqrt(deg)
    xw = jnp.dot(x_ref[...], w_ref[...], preferred_element_type=jnp.float32)
    z_ref[...] = xw * dis
    dis_ref[...] = jnp.broadcast_to(dis, (_BR, _H))


_prep = pl.pallas_call(
    _prep_body,
    grid=(_N // _BR,),
    in_specs=[
        pl.BlockSpec((_BR, _H), lambda i: (i, 0)),
        pl.BlockSpec((_H, _H), lambda i: (0, 0)),
        pl.BlockSpec((_BR, _NW), lambda i: (i, 0)),
    ],
    out_specs=[
        pl.BlockSpec((_BR, _H), lambda i: (i, 0)),
        pl.BlockSpec((_BR, _H), lambda i: (i, 0)),
    ],
    out_shape=[
        jax.ShapeDtypeStruct((_N, _H), jnp.float32),
        jax.ShapeDtypeStruct((_N, _H), jnp.float32),
    ],
)


def _fuse_body(p_ref, z_ref, dis_ref, b_ref, w_ref, zn_ref):
    x = jnp.maximum(
        dis_ref[...] * (p_ref[0] + p_ref[1] + z_ref[...]) + b_ref[...], 0.0)
    zn_ref[...] = jnp.dot(
        x, w_ref[...], preferred_element_type=jnp.float32) * dis_ref[...]


_fuse = pl.pallas_call(
    _fuse_body,
    grid=(_N // _BR,),
    in_specs=[
        pl.BlockSpec((_NC, _BR, _H), lambda i: (0, i, 0)),
        pl.BlockSpec((_BR, _H), lambda i: (i, 0)),
        pl.BlockSpec((_BR, _H), lambda i: (i, 0)),
        pl.BlockSpec((1, _H), lambda i: (0, 0)),
        pl.BlockSpec((_H, _H), lambda i: (0, 0)),
    ],
    out_specs=pl.BlockSpec((_BR, _H), lambda i: (i, 0)),
    out_shape=jax.ShapeDtypeStruct((_N, _H), jnp.float32),
)


def _emean_body(ea_ref, o_ref):
    o_ref[...] = jnp.sum(ea_ref[...], axis=0, keepdims=True)[None]


_emean = pl.pallas_call(
    _emean_body,
    grid=(8,),
    in_specs=[pl.BlockSpec((_E * _ED // 128 // 8, 128), lambda i: (i, 0))],
    out_specs=pl.BlockSpec((1, 1, 128), lambda i: (i, 0, 0)),
    out_shape=jax.ShapeDtypeStruct((8, 1, 128), jnp.float32),
)


_CPOOL = 256
_NPAD = _N + _CPOOL


def _head_body(p_ref, z_ref, dis_ref, b2_ref, em_ref, we_ref, be_ref,
               batch_ref, bbuf_ref, wl1_ref, bl1_ref, wl2_ref, bl2_ref,
               o_ref, xbuf):
    s128 = jnp.sum(em_ref[...], axis=0)
    m16 = s128[0:16]
    for j in range(1, 8):
        m16 = m16 + s128[j * 16:(j + 1) * 16]
    m16 = m16 * (1.0 / _E)
    mef = jnp.sum(we_ref[...] * m16[:, None], axis=0) + be_ref[0]
    mef2 = mef[None, :]

    for i in range(_N // _BR):
        sl = pl.ds(i * _BR, _BR)
        xb = jnp.maximum(
            dis_ref[sl] * (p_ref[0, sl] + p_ref[1, sl] + z_ref[sl])
            + b2_ref[...], 0.0) + mef2
        xbuf[sl] = xb

    batv = batch_ref[...]
    neg = jnp.float32(-3.0e38)
    gi = lax.broadcasted_iota(jnp.int32, (_G, 1), 0)

    def gbody(g, carry):
        start, pooled = carry
        cnt = jnp.sum(jnp.where(batv == g, 1, 0))
        end = start + cnt

        def cond(st):
            return st[0] < end

        def cbody(st):
            off, mx, sm = st
            offh = pl.multiple_of(off, _CPOOL)
            chunk = xbuf[pl.ds(offh, _CPOOL)]
            m = bbuf_ref[pl.ds(offh, _CPOOL)] == g
            mx = jnp.maximum(mx, jnp.where(m, chunk, neg))
            sm = sm + jnp.where(m, chunk, 0.0)
            return (off + _CPOOL, mx, sm)

        st0 = ((start // _CPOOL) * _CPOOL,
               jnp.full((_CPOOL, _H), neg, jnp.float32),
               jnp.zeros((_CPOOL, _H), jnp.float32))
        _, mx, sm = lax.while_loop(cond, cbody, st0)
        mxr = jnp.max(mx, axis=0, keepdims=True)
        smr = jnp.sum(sm, axis=0, keepdims=True)
        mxr = jnp.where(cnt > 0, mxr, 0.0)
        mnr = smr / jnp.maximum(cnt.astype(jnp.float32), 1.0)
        row = jnp.concatenate([mxr, mnr], axis=1)
        pooled = jnp.where(gi == g, row, pooled)
        return (end, pooled)

    _, pooled = lax.fori_loop(
        0, _G, gbody,
        (jnp.int32(0), jnp.zeros((_G, 2 * _H), jnp.float32)))

    h = jnp.maximum(
        jnp.dot(pooled, wl1_ref[...], preferred_element_type=jnp.float32)
        + bl1_ref[...], 0.0)
    o_ref[...] = jnp.dot(
        h, wl2_ref[...], preferred_element_type=jnp.float32) + bl2_ref[...]


_head = pl.pallas_call(
    _head_body,
    out_shape=jax.ShapeDtypeStruct((_G, _OUT), jnp.float32),
    scratch_shapes=[
        pltpu.VMEM((_NPAD, _H), jnp.float32),
    ],
)


def kernel(x, edge_index, edge_attr, batch,
           Wc0, bc0, Wc1, bc1, Wc2, bc2, We, be, Wl1, bl1, Wl2, bl2):
    pad = _EPAD - _E
    rowp = jnp.concatenate(
        [edge_index[0], jnp.zeros((pad,), jnp.int32)]).reshape(_NW, _CH, _K)
    colp = jnp.concatenate(
        [edge_index[1], jnp.full((pad,), _N, jnp.int32)]
    ).reshape(_NW, _CH, _K)
    zeroH = jnp.zeros((_ZR, _H), jnp.float32)
    ea2d = edge_attr.reshape(_E * _ED // 128, 128)
    batch2d = batch.reshape(80, 125)
    bpad = jnp.concatenate([batch, jnp.full((_NPAD - _N,), _G, jnp.int32)])
    bbuf2d = jnp.broadcast_to(bpad[:, None], (_NPAD, _H))

    d = _deg_pass()(colp).T
    z0, disb = _prep(x, Wc0, d)
    s0 = _edge_pass()(z0, rowp, colp, zeroH)
    z1 = _fuse(s0, z0, disb, bc0.reshape(1, _H), Wc1)
    s1 = _edge_pass()(z1, rowp, colp, zeroH)
    z2 = _fuse(s1, z1, disb, bc1.reshape(1, _H), Wc2)
    s2 = _edge_pass()(z2, rowp, colp, zeroH)
    em8 = _emean(ea2d).reshape(8, 128)
    out = _head(s2, z2, disb, bc2.reshape(1, _H), em8, We,
                be.reshape(1, _H), batch2d, bbuf2d,
                Wl1, bl1.reshape(1, _H // 2), Wl2, bl2.reshape(1, _OUT))
    return out


# speedup vs baseline: 1.2565x; 1.2565x over previous
import functools

import jax
import jax.numpy as jnp
from jax import lax
from jax.experimental import pallas as pl
from jax.experimental.pallas import tpu as pltpu
from jax.experimental.pallas import tpu_sc as plsc

_N = 10000
_E = 320000
_H = 128
_G = 128
_ED = 16
_OUT = 32

_NC = 2
_NS = 16
_NW = _NC * _NS
_K = 128
_CH = 80
_EPAD = _NW * _CH * _K
_NACC = 10112
_ZR = _NACC // _NS
_BR = 2000


def _sc_mesh():
    return plsc.VectorSubcoreMesh(
        core_axis_name="c", subcore_axis_name="s",
        num_cores=_NC, num_subcores=_NS)


def _deg_body(col_hbm, out_hbm, col_v, hist):
    cid = lax.axis_index("c")
    sid = lax.axis_index("s")
    wid = cid * _NS + sid
    pltpu.sync_copy(col_hbm.at[wid], col_v)
    zero16 = jnp.zeros((16,), jnp.float32)
    one16 = jnp.ones((16,), jnp.float32)

    def zbody(i, c):
        hist[pl.ds(i * 16, 16)] = zero16
        return c

    lax.fori_loop(0, _NACC // 16, zbody, 0)

    def body(j, c):
        def inner(k, c2):
            idx = col_v[j, pl.ds(k * 16, 16)]
            plsc.addupdate_scatter(hist, [idx], one16)
            return c2

        lax.fori_loop(0, _K // 16, inner, 0)
        return c

    lax.fori_loop(0, _CH, body, 0)
    pltpu.sync_copy(hist, out_hbm.at[wid])


@functools.cache
def _deg_pass():
    return pl.kernel(
        _deg_body,
        out_type=jax.ShapeDtypeStruct((_NW, _NACC), jnp.float32),
        mesh=_sc_mesh(),
        compiler_params=pltpu.CompilerParams(needs_layout_passes=False),
        scratch_types=[
            pltpu.VMEM((_CH, _K), jnp.int32),
            pltpu.VMEM((_NACC,), jnp.float32),
        ],
    )


def _edge_body(z_hbm, row_hbm, col_hbm, zero_hbm, out_hbm,
               row_v, col_v, buf, acc, sem):
    cid = lax.axis_index("c")
    sid = lax.axis_index("s")
    wid = cid * _NS + sid
    pltpu.sync_copy(zero_hbm, acc.at[pl.ds(sid * _ZR, _ZR)])
    pltpu.sync_copy(row_hbm.at[wid], row_v)
    pltpu.sync_copy(col_hbm.at[wid], col_v)
    plsc.subcore_barrier()

    def body(j, carry):
        pltpu.async_copy(z_hbm.at[row_v.at[j]], buf, sem).wait()
        pltpu.sync_copy(buf, acc.at[col_v.at[j]], add=True)
        return carry

    lax.fori_loop(0, _CH, body, 0)
    plsc.subcore_barrier()
    pltpu.sync_copy(acc.at[pl.ds(sid * _ZR, _ZR)],
                    out_hbm.at[cid].at[pl.ds(sid * _ZR, _ZR)])


@functools.cache
def _edge_pass():
    return pl.kernel(
        _edge_body,
        out_type=jax.ShapeDtypeStruct((_NC, _NACC, _H), jnp.float32),
        mesh=_sc_mesh(),
        scratch_types=[
            pltpu.VMEM((_CH, _K), jnp.int32),
            pltpu.VMEM((_CH, _K), jnp.int32),
            pltpu.VMEM((_K, _H), jnp.float32),
            pltpu.VMEM_SHARED((_NACC, _H), jnp.float32),
            pltpu.SemaphoreType.DMA,
        ],
    )


def _prep_body(x_ref, w_ref, d_ref, z_ref, dis_ref):
    deg = jnp.sum(d_ref[...], axis=1, keepdims=True) + 1.0
    dis = lax.rsqrt(deg)
    xw = jnp.dot(x_ref[...], w_ref[...], preferred_element_type=jnp.float32)
    z_ref[...] = xw * dis
    dis_ref[...] = jnp.broadcast_to(dis, (_BR, _H))


_prep = pl.pallas_call(
    _prep_body,
    grid=(_N // _BR,),
    in_specs=[
        pl.BlockSpec((_BR, _H), lambda i: (i, 0)),
        pl.BlockSpec((_H, _H), lambda i: (0, 0)),
        pl.BlockSpec((_BR, _NW), lambda i: (i, 0)),
    ],
    out_specs=[
        pl.BlockSpec((_BR, _H), lambda i: (i, 0)),
        pl.BlockSpec((_BR, _H), lambda i: (i, 0)),
    ],
    out_shape=[
        jax.ShapeDtypeStruct((_N, _H), jnp.float32),
        jax.ShapeDtypeStruct((_N, _H), jnp.float32),
    ],
)


def _fuse_body(p_ref, z_ref, dis_ref, b_ref, w_ref, zn_ref):
    x = jnp.maximum(
        dis_ref[...] * (p_ref[0] + p_ref[1] + z_ref[...]) + b_ref[...], 0.0)
    zn_ref[...] = jnp.dot(
        x, w_ref[...], preferred_element_type=jnp.float32) * dis_ref[...]


_fuse = pl.pallas_call(
    _fuse_body,
    grid=(_N // _BR,),
    in_specs=[
        pl.BlockSpec((_NC, _BR, _H), lambda i: (0, i, 0)),
        pl.BlockSpec((_BR, _H), lambda i: (i, 0)),
        pl.BlockSpec((_BR, _H), lambda i: (i, 0)),
        pl.BlockSpec((1, _H), lambda i: (0, 0)),
        pl.BlockSpec((_H, _H), lambda i: (0, 0)),
    ],
    out_specs=pl.BlockSpec((_BR, _H), lambda i: (i, 0)),
    out_shape=jax.ShapeDtypeStruct((_N, _H), jnp.float32),
)


def _emean_body(ea_ref, o_ref):
    o_ref[...] = jnp.sum(ea_ref[...], axis=0, keepdims=True)[None]


_emean = pl.pallas_call(
    _emean_body,
    grid=(8,),
    in_specs=[pl.BlockSpec((_E * _ED // 128 // 8, 128), lambda i: (i, 0))],
    out_specs=pl.BlockSpec((1, 1, 128), lambda i: (i, 0, 0)),
    out_shape=jax.ShapeDtypeStruct((8, 1, 128), jnp.float32),
)


_CPOOL = 256
_NPAD = _N + _CPOOL


def _head_body(p_ref, z_ref, dis_ref, b2_ref, em_ref, we_ref, be_ref,
               batch_ref, bbuf_ref, wl1_ref, bl1_ref, wl2_ref, bl2_ref,
               o_ref, xbuf):
    s128 = jnp.sum(em_ref[...], axis=0)
    m16 = s128[0:16]
    for j in range(1, 8):
        m16 = m16 + s128[j * 16:(j + 1) * 16]
    m16 = m16 * (1.0 / _E)
    mef = jnp.sum(we_ref[...] * m16[:, None], axis=0) + be_ref[0]
    mef2 = mef[None, :]

    for i in range(_N // _BR):
        sl = pl.ds(i * _BR, _BR)
        xb = jnp.maximum(
            dis_ref[sl] * (p_ref[0, sl] + p_ref[1, sl] + z_ref[sl])
            + b2_ref[...], 0.0) + mef2
        xbuf[sl] = xb

    batv = batch_ref[...]
    neg = jnp.float32(-3.0e38)
    gi = lax.broadcasted_iota(jnp.int32, (_G, 1), 0)

    def gbody(g, carry):
        start, pooled = carry
        cnt = jnp.sum(jnp.where(batv == g, 1, 0))
        end = start + cnt

        def cond(st):
            return st[0] < end

        def cbody(st):
            off, mx, sm = st
            offh = pl.multiple_of(off, _CPOOL)
            chunk = xbuf[pl.ds(offh, _CPOOL)]
            m = bbuf_ref[pl.ds(offh, _CPOOL)] == g
            mx = jnp.maximum(mx, jnp.where(m, chunk, neg))
            sm = sm + jnp.where(m, chunk, 0.0)
            return (off + _CPOOL, mx, sm)

        st0 = ((start // _CPOOL) * _CPOOL,
               jnp.full((_CPOOL, _H), neg, jnp.float32),
               jnp.zeros((_CPOOL, _H), jnp.float32))
        _, mx, sm = lax.while_loop(cond, cbody, st0)
        mxr = jnp.max(mx, axis=0, keepdims=True)
        smr = jnp.sum(sm, axis=0, keepdims=True)
        mxr = jnp.where(cnt > 0, mxr, 0.0)
        mnr = smr / jnp.maximum(cnt.astype(jnp.float32), 1.0)
        row = jnp.concatenate([mxr, mnr], axis=1)
        pooled = jnp.where(gi == g, row, pooled)
        return (end, pooled)

    _, pooled = lax.fori_loop(
        0, _G, gbody,
        (jnp.int32(0), jnp.zeros((_G, 2 * _H), jnp.float32)))

    h = jnp.maximum(
        jnp.dot(pooled, wl1_ref[...], preferred_element_type=jnp.float32)
        + bl1_ref[...], 0.0)
    o_ref[...] = jnp.dot(
        h, wl2_ref[...], preferred_element_type=jnp.float32) + bl2_ref[...]


_head = pl.pallas_call(
    _head_body,
    out_shape=jax.ShapeDtypeStruct((_G, _OUT), jnp.float32),
    scratch_shapes=[
        pltpu.VMEM((_NPAD, _H), jnp.float32),
    ],
)


def kernel(x, edge_index, edge_attr, batch,
           Wc0, bc0, Wc1, bc1, Wc2, bc2, We, be, Wl1, bl1, Wl2, bl2):
    epw = _E // _NW
    padw = _CH * _K - epw
    rows32 = edge_index[0].reshape(_NW, epw)
    cols32 = edge_index[1].reshape(_NW, epw)
    padrow = jnp.zeros((_NW, padw), jnp.int32)
    padcol = jnp.broadcast_to(
        _N + jnp.arange(padw, dtype=jnp.int32) % (_NACC - _N), (_NW, padw))
    rowp = jnp.concatenate([rows32, padrow], axis=1).reshape(_NW, _CH, _K)
    colp = jnp.concatenate([cols32, padcol], axis=1).reshape(_NW, _CH, _K)
    zeroH = jnp.zeros((_ZR, _H), jnp.float32)
    ea2d = edge_attr.reshape(_E * _ED // 128, 128)
    batch2d = batch.reshape(80, 125)
    bpad = jnp.concatenate([batch, jnp.full((_NPAD - _N,), _G, jnp.int32)])
    bbuf2d = jnp.broadcast_to(bpad[:, None], (_NPAD, _H))

    d = _deg_pass()(colp).T
    z0, disb = _prep(x, Wc0, d)
    s0 = _edge_pass()(z0, rowp, colp, zeroH)
    z1 = _fuse(s0, z0, disb, bc0.reshape(1, _H), Wc1)
    s1 = _edge_pass()(z1, rowp, colp, zeroH)
    z2 = _fuse(s1, z1, disb, bc1.reshape(1, _H), Wc2)
    s2 = _edge_pass()(z2, rowp, colp, zeroH)
    em8 = _emean(ea2d).reshape(8, 128)
    out = _head(s2, z2, disb, bc2.reshape(1, _H), em8, We,
                be.reshape(1, _H), batch2d, bbuf2d,
                Wl1, bl1.reshape(1, _H // 2), Wl2, bl2.reshape(1, _OUT))
    return out


# speedup vs baseline: 1.5588x; 1.2406x over previous
import functools

import jax
import jax.numpy as jnp
from jax import lax
from jax.experimental import pallas as pl
from jax.experimental.pallas import tpu as pltpu
from jax.experimental.pallas import tpu_sc as plsc

_N = 10000
_E = 320000
_H = 128
_G = 128
_ED = 16
_OUT = 32

_NC = 2
_NS = 16
_NW = _NC * _NS
_K = 128
_CH = 79
_EPAD = _NW * _CH * _K
_NACC = 10112
_ZR = _NACC // _NS
_BR = 2000


def _sc_mesh():
    return plsc.VectorSubcoreMesh(
        core_axis_name="c", subcore_axis_name="s",
        num_cores=_NC, num_subcores=_NS)


def _deg_body(col_hbm, out_hbm, col_v, hist):
    cid = lax.axis_index("c")
    sid = lax.axis_index("s")
    wid = cid * _NS + sid
    pltpu.sync_copy(col_hbm.at[wid], col_v)
    zero16 = jnp.zeros((16,), jnp.float32)
    one16 = jnp.ones((16,), jnp.float32)

    def zbody(i, c):
        hist[pl.ds(i * 16, 16)] = zero16
        return c

    lax.fori_loop(0, _NACC // 16, zbody, 0)

    def body(j, c):
        def inner(k, c2):
            idx = col_v[j, pl.ds(k * 16, 16)]
            plsc.addupdate_scatter(hist, [idx], one16)
            return c2

        lax.fori_loop(0, _K // 16, inner, 0)
        return c

    lax.fori_loop(0, _CH, body, 0)
    pltpu.sync_copy(hist, out_hbm.at[wid])


@functools.cache
def _deg_pass():
    return pl.kernel(
        _deg_body,
        out_type=jax.ShapeDtypeStruct((_NW, _NACC), jnp.float32),
        mesh=_sc_mesh(),
        compiler_params=pltpu.CompilerParams(needs_layout_passes=False),
        scratch_types=[
            pltpu.VMEM((_CH, _K), jnp.int32),
            pltpu.VMEM((_NACC,), jnp.float32),
        ],
    )


def _edge_body(z_hbm, row_hbm, col_hbm, zero_hbm, out_hbm,
               row_v, col_v, buf, acc, sem):
    cid = lax.axis_index("c")
    sid = lax.axis_index("s")
    wid = cid * _NS + sid
    pltpu.sync_copy(zero_hbm, acc.at[pl.ds(sid * _ZR, _ZR)])
    pltpu.sync_copy(row_hbm.at[wid], row_v)
    pltpu.sync_copy(col_hbm.at[wid], col_v)
    plsc.subcore_barrier()

    def body(j, carry):
        pltpu.async_copy(z_hbm.at[row_v.at[j]], buf, sem).wait()
        pltpu.sync_copy(buf, acc.at[col_v.at[j]], add=True)
        return carry

    lax.fori_loop(0, _CH, body, 0)
    plsc.subcore_barrier()
    pltpu.sync_copy(acc.at[pl.ds(sid * _ZR, _ZR)],
                    out_hbm.at[cid].at[pl.ds(sid * _ZR, _ZR)])


@functools.cache
def _edge_pass():
    return pl.kernel(
        _edge_body,
        out_type=jax.ShapeDtypeStruct((_NC, _NACC, _H), jnp.float32),
        mesh=_sc_mesh(),
        scratch_types=[
            pltpu.VMEM((_CH, _K), jnp.int32),
            pltpu.VMEM((_CH, _K), jnp.int32),
            pltpu.VMEM((_K, _H), jnp.float32),
            pltpu.VMEM_SHARED((_NACC, _H), jnp.float32),
            pltpu.SemaphoreType.DMA,
        ],
    )


def _prep_body(x_ref, w_ref, d_ref, z_ref, dis_ref):
    deg = jnp.sum(d_ref[...], axis=1, keepdims=True) + 1.0
    dis = lax.rsqrt(deg)
    xw = jnp.dot(x_ref[...], w_ref[...], preferred_element_type=jnp.float32)
    z_ref[...] = xw * dis
    dis_ref[...] = jnp.broadcast_to(dis, (_BR, _H))


_prep = pl.pallas_call(
    _prep_body,
    grid=(_N // _BR,),
    in_specs=[
        pl.BlockSpec((_BR, _H), lambda i: (i, 0)),
        pl.BlockSpec((_H, _H), lambda i: (0, 0)),
        pl.BlockSpec((_BR, _NW), lambda i: (i, 0)),
    ],
    out_specs=[
        pl.BlockSpec((_BR, _H), lambda i: (i, 0)),
        pl.BlockSpec((_BR, _H), lambda i: (i, 0)),
    ],
    out_shape=[
        jax.ShapeDtypeStruct((_N, _H), jnp.float32),
        jax.ShapeDtypeStruct((_N, _H), jnp.float32),
    ],
)


def _fuse_body(p_ref, z_ref, dis_ref, b_ref, w_ref, zn_ref):
    x = jnp.maximum(
        dis_ref[...] * (p_ref[0] + p_ref[1] + z_ref[...]) + b_ref[...], 0.0)
    zn_ref[...] = jnp.dot(
        x, w_ref[...], preferred_element_type=jnp.float32) * dis_ref[...]


_fuse = pl.pallas_call(
    _fuse_body,
    grid=(_N // _BR,),
    in_specs=[
        pl.BlockSpec((_NC, _BR, _H), lambda i: (0, i, 0)),
        pl.BlockSpec((_BR, _H), lambda i: (i, 0)),
        pl.BlockSpec((_BR, _H), lambda i: (i, 0)),
        pl.BlockSpec((1, _H), lambda i: (0, 0)),
        pl.BlockSpec((_H, _H), lambda i: (0, 0)),
    ],
    out_specs=pl.BlockSpec((_BR, _H), lambda i: (i, 0)),
    out_shape=jax.ShapeDtypeStruct((_N, _H), jnp.float32),
)


def _emean_body(ea_ref, o_ref):
    o_ref[...] = jnp.sum(ea_ref[...], axis=0, keepdims=True)[None]


_emean = pl.pallas_call(
    _emean_body,
    grid=(8,),
    in_specs=[pl.BlockSpec((_E * _ED // 128 // 8, 128), lambda i: (i, 0))],
    out_specs=pl.BlockSpec((1, 1, 128), lambda i: (i, 0, 0)),
    out_shape=jax.ShapeDtypeStruct((8, 1, 128), jnp.float32),
)


_CPOOL = 256
_NPAD = _N + _CPOOL


def _head_body(p_ref, z_ref, dis_ref, b2_ref, em_ref, we_ref, be_ref,
               batch_ref, bbuf_ref, wl1_ref, bl1_ref, wl2_ref, bl2_ref,
               o_ref, xbuf):
    s128 = jnp.sum(em_ref[...], axis=0)
    m16 = s128[0:16]
    for j in range(1, 8):
        m16 = m16 + s128[j * 16:(j + 1) * 16]
    m16 = m16 * (1.0 / _E)
    mef = jnp.sum(we_ref[...] * m16[:, None], axis=0) + be_ref[0]
    mef2 = mef[None, :]

    for i in range(_N // _BR):
        sl = pl.ds(i * _BR, _BR)
        xb = jnp.maximum(
            dis_ref[sl] * (p_ref[0, sl] + p_ref[1, sl] + z_ref[sl])
            + b2_ref[...], 0.0) + mef2
        xbuf[sl] = xb

    batv = batch_ref[...]
    neg = jnp.float32(-3.0e38)
    gi = lax.broadcasted_iota(jnp.int32, (_G, 1), 0)

    def gbody(g, carry):
        start, pooled = carry
        cnt = jnp.sum(jnp.where(batv == g, 1, 0))
        end = start + cnt

        def cond(st):
            return st[0] < end

        def cbody(st):
            off, mx, sm = st
            offh = pl.multiple_of(off, _CPOOL)
            chunk = xbuf[pl.ds(offh, _CPOOL)]
            m = bbuf_ref[pl.ds(offh, _CPOOL)] == g
            mx = jnp.maximum(mx, jnp.where(m, chunk, neg))
            sm = sm + jnp.where(m, chunk, 0.0)
            return (off + _CPOOL, mx, sm)

        st0 = ((start // _CPOOL) * _CPOOL,
               jnp.full((_CPOOL, _H), neg, jnp.float32),
               jnp.zeros((_CPOOL, _H), jnp.float32))
        _, mx, sm = lax.while_loop(cond, cbody, st0)
        mxr = jnp.max(mx, axis=0, keepdims=True)
        smr = jnp.sum(sm, axis=0, keepdims=True)
        mxr = jnp.where(cnt > 0, mxr, 0.0)
        mnr = smr / jnp.maximum(cnt.astype(jnp.float32), 1.0)
        row = jnp.concatenate([mxr, mnr], axis=1)
        pooled = jnp.where(gi == g, row, pooled)
        return (end, pooled)

    _, pooled = lax.fori_loop(
        0, _G, gbody,
        (jnp.int32(0), jnp.zeros((_G, 2 * _H), jnp.float32)))

    h = jnp.maximum(
        jnp.dot(pooled, wl1_ref[...], preferred_element_type=jnp.float32)
        + bl1_ref[...], 0.0)
    o_ref[...] = jnp.dot(
        h, wl2_ref[...], preferred_element_type=jnp.float32) + bl2_ref[...]


_head = pl.pallas_call(
    _head_body,
    out_shape=jax.ShapeDtypeStruct((_G, _OUT), jnp.float32),
    scratch_shapes=[
        pltpu.VMEM((_NPAD, _H), jnp.float32),
    ],
)


def kernel(x, edge_index, edge_attr, batch,
           Wc0, bc0, Wc1, bc1, Wc2, bc2, We, be, Wl1, bl1, Wl2, bl2):
    pad = _EPAD - _E
    rowp = jnp.concatenate(
        [edge_index[0], jnp.zeros((pad,), jnp.int32)]).reshape(_NW, _CH, _K)
    colp = jnp.concatenate(
        [edge_index[1],
         _N + jnp.arange(pad, dtype=jnp.int32) % (_NACC - _N)]
    ).reshape(_NW, _CH, _K)
    zeroH = jnp.zeros((_ZR, _H), jnp.float32)
    ea2d = edge_attr.reshape(_E * _ED // 128, 128)
    batch2d = batch.reshape(80, 125)
    bpad = jnp.concatenate([batch, jnp.full((_NPAD - _N,), _G, jnp.int32)])
    bbuf2d = jnp.broadcast_to(bpad[:, None], (_NPAD, _H))

    d = _deg_pass()(colp).T
    z0, disb = _prep(x, Wc0, d)
    s0 = _edge_pass()(z0, rowp, colp, zeroH)
    z1 = _fuse(s0, z0, disb, bc0.reshape(1, _H), Wc1)
    s1 = _edge_pass()(z1, rowp, colp, zeroH)
    z2 = _fuse(s1, z1, disb, bc1.reshape(1, _H), Wc2)
    s2 = _edge_pass()(z2, rowp, colp, zeroH)
    em8 = _emean(ea2d).reshape(8, 128)
    out = _head(s2, z2, disb, bc2.reshape(1, _H), em8, We,
                be.reshape(1, _H), batch2d, bbuf2d,
                Wl1, bl1.reshape(1, _H // 2), Wl2, bl2.reshape(1, _OUT))
    return out
